# Initial kernel scaffold; baseline (speedup 1.0000x reference)
import jax
import jax.numpy as jnp
from jax.experimental import pallas as pl


def kernel(row_embed, col_embed, bev_h, bev_w):
    raise NotImplementedError("write your pallas kernel here")


# speedup vs baseline: 3.0101x; 3.0101x over previous
import jax
import jax.numpy as jnp
from jax.experimental import pallas as pl


def _pos_body(row_ref, col_ref, out_ref):
    r, nf = row_ref.shape
    w = col_ref.shape[0]
    col = col_ref[...]
    row = row_ref[...]
    out_ref[:, :, 0:nf] = jnp.broadcast_to(col[None, :, :], (r, w, nf))
    out_ref[:, :, nf : 2 * nf] = jnp.broadcast_to(row[:, None, :], (r, w, nf))


def kernel(row_embed, col_embed, bev_h, bev_w):
    h, nf = row_embed.shape
    w, _ = col_embed.shape
    r = 40
    out = pl.pallas_call(
        _pos_body,
        grid=(h // r,),
        in_specs=[
            pl.BlockSpec((r, nf), lambda i: (i, 0)),
            pl.BlockSpec((w, nf), lambda i: (0, 0)),
        ],
        out_specs=pl.BlockSpec((r, w, 2 * nf), lambda i: (i, 0, 0)),
        out_shape=jax.ShapeDtypeStruct((h, w, 2 * nf), jnp.float32),
    )(row_embed, col_embed)
    return out.reshape(1, h * w, 2 * nf)
